# baseline (speedup 1.0000x reference)
import functools
import math

import jax
import jax.numpy as jnp
from jax import lax
from jax.experimental import pallas as pl
from jax.experimental.pallas import tpu as pltpu
from jax.experimental.pallas import tpu_sc as plsc


_TT = 512


def _score_body(x_ref, w_ref, o_ref):
    o_ref[...] = jnp.dot(x_ref[...], w_ref[...],
                         preferred_element_type=jnp.float32)


def _scores(x2, w2):
    nbt = x2.shape[0] // _TT
    d = x2.shape[1]
    return pl.pallas_call(
        _score_body,
        grid=(nbt,),
        in_specs=[
            pl.BlockSpec((_TT, d), lambda i: (i, 0)),
            pl.BlockSpec((d, 1), lambda i: (0, 0)),
        ],
        out_specs=pl.BlockSpec((_TT, 1), lambda i: (i, 0)),
        out_shape=jax.ShapeDtypeStruct((x2.shape[0], 1), jnp.float32),
    )(x2, w2)


_JC = 512
_KC = 512


def _sortable(v):
    u = lax.bitcast_convert_type(v, jnp.int32)
    return u ^ (lax.shift_right_arithmetic(u, 31) & jnp.int32(0x7FFFFFFF))


def _topk_body(scol_ref, srow_ref, idx_ref, gidx_ref):
    b_sz, t = srow_ref.shape
    k = idx_ref.shape[1]
    for b in range(b_sz):
        ks_col = _sortable(scol_ref[b * t:(b + 1) * t, :])
        acc = jnp.zeros((t, 1), jnp.int32)
        for jc in range(t // _JC):
            ks_row = _sortable(srow_ref[b:b + 1, jc * _JC:(jc + 1) * _JC])
            gt = ks_row > ks_col
            eq = ks_row == ks_col
            jglob = lax.broadcasted_iota(jnp.int32, (t, _JC), 1) + jc * _JC
            iglob = lax.broadcasted_iota(jnp.int32, (t, _JC), 0)
            beats = gt | (eq & (jglob < iglob))
            acc = acc + jnp.sum(beats.astype(jnp.int32), axis=1,
                                keepdims=True)
        for kc in range(k // _KC):
            rvals = lax.broadcasted_iota(jnp.int32, (t, _KC), 1) + kc * _KC
            ig = lax.broadcasted_iota(jnp.int32, (t, _KC), 0)
            hit = acc == rvals
            contrib = jnp.sum(jnp.where(hit, ig, 0), axis=0)
            idx_ref[b, kc * _KC:(kc + 1) * _KC] = contrib
            gidx_ref[b, kc * _KC:(kc + 1) * _KC] = contrib + b * t


def _topk(scol, srow, k):
    b, t = srow.shape
    return pl.pallas_call(
        _topk_body,
        out_shape=(
            jax.ShapeDtypeStruct((b, k), jnp.int32),
            jax.ShapeDtypeStruct((b, k), jnp.int32),
        ),
    )(scol, srow)


_CH = 16


def _make_sc_gather(n_rows, d):
    info = plsc.get_sparse_core_info()
    nw = info.num_cores * info.num_subcores
    nc = info.num_cores
    b_per_w = n_rows // nw
    n_ch = b_per_w // _CH
    mesh = plsc.VectorSubcoreMesh(core_axis_name="c", subcore_axis_name="s")

    @functools.partial(
        pl.kernel,
        mesh=mesh,
        out_type=jax.ShapeDtypeStruct((n_rows, d), jnp.float32),
        scratch_types=[
            pltpu.VMEM((_CH,), jnp.int32),
            pltpu.VMEM((_CH, d), jnp.float32),
            pltpu.SemaphoreType.DMA,
        ],
    )
    def gather_k(table_hbm, idx_hbm, out_hbm, idx_v, rows_v, sem):
        wid = lax.axis_index("s") * nc + lax.axis_index("c")
        base = wid * b_per_w
        for c in range(n_ch):
            off = base + c * _CH
            pltpu.sync_copy(idx_hbm.at[pl.ds(off, _CH)], idx_v)
            pltpu.async_copy(table_hbm.at[idx_v], rows_v, sem).wait()
            pltpu.sync_copy(rows_v, out_hbm.at[pl.ds(off, _CH)])

    return gather_k


def kernel(x, W):
    b, t, d = x.shape
    k = max(1, math.ceil(0.5 * t))

    x2 = x.reshape(b * t, d)
    scol = _scores(x2, W.reshape(d, 1))
    srow = scol.reshape(b, t)
    indices, gidx = _topk(scol, srow, k)

    gather_fn = _make_sc_gather(b * k, d)
    selected = gather_fn(x2, gidx.reshape(b * k))
    return selected.reshape(b, k, d), indices, srow


# speedup vs baseline: 1.1894x; 1.1894x over previous
import functools
import math

import jax
import jax.numpy as jnp
from jax import lax
from jax.experimental import pallas as pl
from jax.experimental.pallas import tpu as pltpu
from jax.experimental.pallas import tpu_sc as plsc


_TT = 512


def _score_body(x_ref, w_ref, o_ref):
    o_ref[...] = jnp.dot(x_ref[...], w_ref[...],
                         preferred_element_type=jnp.float32)


def _scores(x2, w2):
    nbt = x2.shape[0] // _TT
    d = x2.shape[1]
    return pl.pallas_call(
        _score_body,
        grid=(nbt,),
        in_specs=[
            pl.BlockSpec((_TT, d), lambda i: (i, 0)),
            pl.BlockSpec((d, 1), lambda i: (0, 0)),
        ],
        out_specs=pl.BlockSpec((_TT, 1), lambda i: (i, 0)),
        out_shape=jax.ShapeDtypeStruct((x2.shape[0], 1), jnp.float32),
    )(x2, w2)


_RC = 512
_KC = 512


def _sortable(v):
    u = lax.bitcast_convert_type(v, jnp.int32)
    return u ^ (lax.shift_right_arithmetic(u, 31) & jnp.int32(0x7FFFFFFF))


def _topk_body(scol_ref, srow_ref, idx_ref, gidx_ref):
    b_sz, t = srow_ref.shape
    k = idx_ref.shape[1]
    ones_col = jnp.ones((t, 1), jnp.float32)
    iota_row = lax.broadcasted_iota(jnp.int32, (1, t), 1).astype(jnp.float32)
    jl_diag = (lax.broadcasted_iota(jnp.int32, (_RC, _RC), 1)
               < lax.broadcasted_iota(jnp.int32, (_RC, _RC), 0))
    for b in range(b_sz):
        ks_col = _sortable(scol_ref[b * t:(b + 1) * t, :])
        ks_row = _sortable(srow_ref[b:b + 1, :])
        acc_blocks = []
        for ic in range(t // _RC):
            lo, hi = ic * _RC, (ic + 1) * _RC
            ks_i = ks_col[lo:hi, :]
            parts = []
            if lo > 0:
                parts.append((ks_row[:, :lo] >= ks_i).astype(jnp.float32))
            ksd = ks_row[:, lo:hi]
            diag = (ksd > ks_i) | ((ksd == ks_i) & jl_diag)
            parts.append(diag.astype(jnp.float32))
            if hi < t:
                parts.append((ks_row[:, hi:] > ks_i).astype(jnp.float32))
            beats = jnp.concatenate(parts, axis=1)
            acc_blocks.append(jnp.dot(beats, ones_col,
                                      preferred_element_type=jnp.float32))
        rank = jnp.concatenate(acc_blocks, axis=0)
        for kc in range(k // _KC):
            rvals = (lax.broadcasted_iota(jnp.int32, (t, _KC), 1)
                     + kc * _KC).astype(jnp.float32)
            hit = (rank == rvals).astype(jnp.float32)
            contrib = jnp.dot(iota_row, hit,
                              preferred_element_type=jnp.float32)
            ci = contrib.astype(jnp.int32)[0, :]
            idx_ref[b, kc * _KC:(kc + 1) * _KC] = ci
            gidx_ref[b, kc * _KC:(kc + 1) * _KC] = ci + b * t


def _topk(scol, srow, k):
    b, t = srow.shape
    return pl.pallas_call(
        _topk_body,
        out_shape=(
            jax.ShapeDtypeStruct((b, k), jnp.int32),
            jax.ShapeDtypeStruct((b, k), jnp.int32),
        ),
    )(scol, srow)


_CH = 16


def _make_sc_gather(n_rows, d):
    info = plsc.get_sparse_core_info()
    nw = info.num_cores * info.num_subcores
    nc = info.num_cores
    b_per_w = n_rows // nw
    n_ch = b_per_w // _CH
    mesh = plsc.VectorSubcoreMesh(core_axis_name="c", subcore_axis_name="s")

    @functools.partial(
        pl.kernel,
        mesh=mesh,
        out_type=jax.ShapeDtypeStruct((n_rows, d), jnp.float32),
        scratch_types=[
            pltpu.VMEM((_CH,), jnp.int32),
            pltpu.VMEM((_CH,), jnp.int32),
            pltpu.VMEM((_CH, d), jnp.float32),
            pltpu.VMEM((_CH, d), jnp.float32),
            pltpu.SemaphoreType.DMA,
            pltpu.SemaphoreType.DMA,
            pltpu.SemaphoreType.DMA,
            pltpu.SemaphoreType.DMA,
        ],
    )
    def gather_k(table_hbm, idx_hbm, out_hbm,
                 idx_v0, idx_v1, rows_v0, rows_v1,
                 sem_g0, sem_g1, sem_o0, sem_o1):
        wid = lax.axis_index("s") * nc + lax.axis_index("c")
        base = wid * b_per_w
        idx_v = [idx_v0, idx_v1]
        rows_v = [rows_v0, rows_v1]
        sem_g = [sem_g0, sem_g1]
        sem_o = [sem_o0, sem_o1]
        g = [None] * n_ch
        w = [None] * n_ch
        for c in range(n_ch):
            p = c % 2
            if c == 0:
                pltpu.sync_copy(idx_hbm.at[pl.ds(base, _CH)], idx_v[0])
                g[0] = pltpu.async_copy(table_hbm.at[idx_v[0]], rows_v[0],
                                        sem_g[0])
            if c + 1 < n_ch:
                pn = (c + 1) % 2
                pltpu.sync_copy(
                    idx_hbm.at[pl.ds(base + (c + 1) * _CH, _CH)], idx_v[pn])
                if c >= 1:
                    w[c - 1].wait()
                g[c + 1] = pltpu.async_copy(table_hbm.at[idx_v[pn]],
                                            rows_v[pn], sem_g[pn])
            g[c].wait()
            w[c] = pltpu.async_copy(
                rows_v[p], out_hbm.at[pl.ds(base + c * _CH, _CH)], sem_o[p])
        if n_ch >= 2:
            w[n_ch - 2].wait()
        w[n_ch - 1].wait()

    return gather_k


def kernel(x, W):
    b, t, d = x.shape
    k = max(1, math.ceil(0.5 * t))

    x2 = x.reshape(b * t, d)
    scol = _scores(x2, W.reshape(d, 1))
    srow = scol.reshape(b, t)
    indices, gidx = _topk(scol, srow, k)

    gather_fn = _make_sc_gather(b * k, d)
    selected = gather_fn(x2, gidx.reshape(b * k))
    return selected.reshape(b, k, d), indices, srow
